# baseline (speedup 1.0000x reference)
import functools

import jax
import jax.numpy as jnp
from jax import lax
from jax.experimental import pallas as pl
from jax.experimental.pallas import tpu as pltpu
from jax.experimental.pallas import tpu_sc as plsc

B = 1024
D = 8
V = 100000

_NC = 2
_NS = 16
_NW = _NC * _NS
_B_PER_W = B // _NW

_TV = 2048
_NBUF = 4
_NSTEP = -(-V // _TV)
_TV_LAST = V - (_NSTEP - 1) * _TV


@functools.lru_cache(maxsize=1)
def _make_sc_gather():
    mesh = plsc.VectorSubcoreMesh(core_axis_name="c", subcore_axis_name="s")

    n_elems = _B_PER_W * D

    @functools.partial(
        pl.kernel,
        mesh=mesh,
        out_type=jax.ShapeDtypeStruct((D + 1, B), jnp.float32),
        scratch_types=[
            pltpu.VMEM((_B_PER_W,), jnp.int32),
            pltpu.VMEM((n_elems,), jnp.int32),
            pltpu.VMEM((n_elems,), jnp.float32),
            pltpu.VMEM((_B_PER_W,), jnp.float32),
            pltpu.SemaphoreType.DMA,
        ],
        compiler_params=pltpu.CompilerParams(
            use_tc_tiling_on_sc=False, needs_layout_passes=False
        ),
    )
    def sc_gather(tflat_hbm, idx_hbm, out_hbm, idx_v, idx8_v, vals_v,
                  ones_v, sem):
        wid = lax.axis_index("s") * _NC + lax.axis_index("c")
        base = wid * _B_PER_W
        pltpu.sync_copy(idx_hbm.at[pl.ds(base, _B_PER_W)], idx_v)
        lanes = lax.iota(jnp.int32, 16)
        ones = jnp.full((16,), 1.0, jnp.float32)
        ones_v[pl.ds(0, 16)] = ones
        ones_v[pl.ds(16, 16)] = ones
        for c in range(n_elems // 16):
            p = lanes + (16 * c)
            row = lax.bitwise_and(p, _B_PER_W - 1)
            dim = lax.shift_right_logical(p, 5)
            xi = plsc.load_gather(idx_v, [row])
            idx8_v[pl.ds(16 * c, 16)] = xi + dim * V
        pltpu.async_copy(tflat_hbm.at[idx8_v], vals_v, sem).wait()
        for d in range(D):
            pltpu.sync_copy(
                vals_v.at[pl.ds(d * _B_PER_W, _B_PER_W)],
                out_hbm.at[d, pl.ds(base, _B_PER_W)],
            )
        pltpu.sync_copy(ones_v, out_hbm.at[D, pl.ds(base, _B_PER_W)])

    return sc_gather


def _mm_body(w_ref, b_ref, e9_ref, out_hbm, buf, sems):
    i = pl.program_id(0)
    j = lax.rem(i, _NBUF)

    def _copy(step, k, n=_TV):
        return pltpu.make_async_copy(
            buf.at[k, pl.ds(0, n)],
            out_hbm.at[pl.ds(step * _TV, n)],
            sems.at[k],
        )

    @pl.when(i >= _NBUF)
    def _():
        _copy(i - _NBUF, j).wait()

    w9 = jnp.concatenate([w_ref[...], b_ref[...]], axis=0)
    buf[j] = lax.dot_general(
        w9,
        e9_ref[...],
        dimension_numbers=(((0,), (0,)), ((), ())),
        preferred_element_type=jnp.float32,
    )

    @pl.when(i < _NSTEP - 1)
    def _():
        _copy(i, j).start()

    @pl.when(i == _NSTEP - 1)
    def _():
        _copy(i, j, _TV_LAST).start()
        for step in range(_NSTEP - _NBUF, _NSTEP):
            _copy(step, step % _NBUF,
                  _TV if step < _NSTEP - 1 else _TV_LAST).wait()


def _tc_project(emb9T, W, b2d):
    return pl.pallas_call(
        _mm_body,
        grid=(_NSTEP,),
        in_specs=[
            pl.BlockSpec((D, _TV), lambda i: (0, i)),
            pl.BlockSpec((1, _TV), lambda i: (0, i)),
            pl.BlockSpec((D + 1, B), lambda i: (0, 0)),
        ],
        out_specs=pl.BlockSpec(memory_space=pl.ANY),
        out_shape=jax.ShapeDtypeStruct((V, B), jnp.float32),
        scratch_shapes=[
            pltpu.VMEM((_NBUF, _TV, B), jnp.float32),
            pltpu.SemaphoreType.DMA((_NBUF,)),
        ],
    )(W, b2d, emb9T)


def kernel(x, E, W, b):
    idx = x.astype(jnp.int32)
    emb9T = _make_sc_gather()(E.T.reshape(-1), idx)
    return _tc_project(emb9T, W, b.reshape(1, V)).T


# speedup vs baseline: 1.0124x; 1.0124x over previous
import functools

import jax
import jax.numpy as jnp
from jax import lax
from jax.experimental import pallas as pl
from jax.experimental.pallas import tpu as pltpu
from jax.experimental.pallas import tpu_sc as plsc

B = 1024
D = 8
V = 100000

_NC = 2
_NS = 16
_NW = _NC * _NS
_B_PER_W = B // _NW

_TV = 2048


@functools.lru_cache(maxsize=1)
def _make_sc_gather():
    mesh = plsc.VectorSubcoreMesh(core_axis_name="c", subcore_axis_name="s")

    n_elems = _B_PER_W * D

    @functools.partial(
        pl.kernel,
        mesh=mesh,
        out_type=jax.ShapeDtypeStruct((D + 1, B), jnp.float32),
        scratch_types=[
            pltpu.VMEM((_B_PER_W,), jnp.int32),
            pltpu.VMEM((n_elems,), jnp.int32),
            pltpu.VMEM((n_elems,), jnp.float32),
            pltpu.VMEM((_B_PER_W,), jnp.float32),
            pltpu.SemaphoreType.DMA,
        ],
        compiler_params=pltpu.CompilerParams(
            use_tc_tiling_on_sc=False, needs_layout_passes=False
        ),
    )
    def sc_gather(tflat_hbm, idx_hbm, out_hbm, idx_v, idx8_v, vals_v,
                  ones_v, sem):
        wid = lax.axis_index("s") * _NC + lax.axis_index("c")
        base = wid * _B_PER_W
        pltpu.sync_copy(idx_hbm.at[pl.ds(base, _B_PER_W)], idx_v)
        lanes = lax.iota(jnp.int32, 16)
        ones = jnp.full((16,), 1.0, jnp.float32)
        ones_v[pl.ds(0, 16)] = ones
        ones_v[pl.ds(16, 16)] = ones
        for c in range(n_elems // 16):
            p = lanes + (16 * c)
            row = lax.bitwise_and(p, _B_PER_W - 1)
            dim = lax.shift_right_logical(p, 5)
            xi = plsc.load_gather(idx_v, [row])
            idx8_v[pl.ds(16 * c, 16)] = xi + dim * V
        pltpu.async_copy(tflat_hbm.at[idx8_v], vals_v, sem).wait()
        for d in range(D):
            pltpu.sync_copy(
                vals_v.at[pl.ds(d * _B_PER_W, _B_PER_W)],
                out_hbm.at[d, pl.ds(base, _B_PER_W)],
            )
        pltpu.sync_copy(ones_v, out_hbm.at[D, pl.ds(base, _B_PER_W)])

    return sc_gather


def _mm_body(w_ref, b_ref, e9_ref, out_ref):
    w9 = jnp.concatenate([w_ref[...], b_ref[...]], axis=0)
    out_ref[...] = lax.dot_general(
        w9,
        e9_ref[...],
        dimension_numbers=(((0,), (0,)), ((), ())),
        preferred_element_type=jnp.float32,
    )


def _tc_project(emb9T, W, b2d):
    return pl.pallas_call(
        _mm_body,
        grid=(pl.cdiv(V, _TV),),
        in_specs=[
            pl.BlockSpec((D, _TV), lambda i: (0, i)),
            pl.BlockSpec((1, _TV), lambda i: (0, i)),
            pl.BlockSpec((D + 1, B), lambda i: (0, 0)),
        ],
        out_specs=pl.BlockSpec((_TV, B), lambda i: (i, 0)),
        out_shape=jax.ShapeDtypeStruct((V, B), jnp.float32),
    )(W, b2d, emb9T)


def kernel(x, E, W, b):
    idx = x.astype(jnp.int32)
    emb9T = _make_sc_gather()(E.T.reshape(-1), idx)
    return _tc_project(emb9T, W, b.reshape(1, V)).T
